# Initial kernel scaffold; baseline (speedup 1.0000x reference)
import jax
import jax.numpy as jnp
from jax.experimental import pallas as pl


def kernel(feats, targets, mem):
    raise NotImplementedError("write your pallas kernel here")


# speedup vs baseline: 8.0584x; 8.0584x over previous
import functools

import jax
import jax.numpy as jnp
from jax import lax
from jax.experimental import pallas as pl
from jax.experimental.pallas import tpu as pltpu
from jax.experimental.pallas import tpu_sc as plsc

B, S, D = 4096, 2, 64
NUM_ID, K = 100000, 4
NC, NS = 2, 16
NW = NC * NS
ROWS_PER_TILE = B // NW
SUB = 32
ROW = S * D
OUT_ROW = (S + K) * D


def _rsqrt(x):
    i = plsc.bitcast(x, jnp.int32)
    y = plsc.bitcast(jnp.int32(0x5F3759DF) - (i >> 1), jnp.float32)
    for _ in range(3):
        y = y * (1.5 - 0.5 * x * y * y)
    return y


def _sc_body(feats_hbm, ids_hbm, f_out, t_out,
             table, ids_all, w_chunk, fs, fw, buf, tbuf, sem):
    wid = lax.axis_index("s") * NC + lax.axis_index("c")
    base = wid * ROWS_PER_TILE

    pltpu.sync_copy(ids_hbm, ids_all)

    lane = lax.iota(jnp.int32, 16)

    def scat_body(k, c):
        j0 = k * 16
        idv = ids_all[pl.ds(j0, 16)]
        key = idv * B + (lane + j0)
        skey = lax.sort(key)
        nxt = jnp.take_along_axis(skey, (lane + 1) & 15, axis=0,
                                  mode="promise_in_bounds")
        last = ((skey // B) != (nxt // B)) | (lane == 15)
        plsc.store_scatter(table, [skey // B], skey % B, mask=last)
        return c
    lax.fori_loop(0, B // 16, scat_body, 0)

    def w_body(k, c):
        idx = ids_all[pl.ds(base + k * 16, 16)]
        w_chunk[pl.ds(k * 16, 16)] = plsc.load_gather(table, [idx])
        return c
    lax.fori_loop(0, ROWS_PER_TILE // 16, w_body, 0)

    def norm_group(vs):
        s = vs[0] * vs[0] + vs[1] * vs[1] + vs[2] * vs[2] + vs[3] * vs[3]
        tot = jnp.sum(s)
        x = jnp.maximum(lax.broadcast(tot, (16,)), 1e-24)
        return _rsqrt(x)

    def q_body(q, c):
        rbase = base + q * SUB
        pltpu.sync_copy(feats_hbm.at[pl.ds(rbase, SUB)], fs)
        pltpu.async_copy(
            feats_hbm.at[w_chunk.at[pl.ds(q * SUB, SUB)]], fw, sem).wait()

        def r_body(r, c2):
            for src, cols in ((fs, (0,)), (fw, (ROW, 2 * ROW))):
                for g in range(S):
                    vs = [src[r, pl.ds(g * D + k2 * 16, 16)]
                          for k2 in range(4)]
                    y = norm_group(vs)
                    for k2 in range(4):
                        o = vs[k2] * y
                        for c0 in cols:
                            buf[r, pl.ds(c0 + g * D + k2 * 16, 16)] = o
            return c2
        lax.fori_loop(0, SUB, r_body, 0)

        for v in range(SUB * (S + K) // 16):
            p = lax.iota(jnp.int32, 16) + v * 16
            tbuf[pl.ds(v * 16, 16)] = plsc.load_gather(
                ids_all, [rbase + p // (S + K)])

        pltpu.sync_copy(buf, f_out.at[pl.ds(rbase, SUB)])
        pltpu.sync_copy(tbuf, t_out.at[pl.ds(rbase * (S + K), SUB * (S + K))])
        return c
    lax.fori_loop(0, ROWS_PER_TILE // SUB, q_body, 0)


@jax.jit
def _sc_call(feats2, ids):
    mesh = plsc.VectorSubcoreMesh(core_axis_name="c", subcore_axis_name="s")
    return pl.kernel(
        _sc_body,
        out_type=(
            jax.ShapeDtypeStruct((B, OUT_ROW), jnp.float32),
            jax.ShapeDtypeStruct((B * (S + K),), jnp.int32),
        ),
        mesh=mesh,
        compiler_params=pltpu.CompilerParams(needs_layout_passes=False),
        scratch_types=[
            pltpu.VMEM((NUM_ID,), jnp.int32),
            pltpu.VMEM((B,), jnp.int32),
            pltpu.VMEM((ROWS_PER_TILE,), jnp.int32),
            pltpu.VMEM((SUB, ROW), jnp.float32),
            pltpu.VMEM((SUB, ROW), jnp.float32),
            pltpu.VMEM((SUB, OUT_ROW), jnp.float32),
            pltpu.VMEM((SUB * (S + K),), jnp.int32),
            pltpu.SemaphoreType.DMA,
        ],
    )(feats2, ids)


def kernel(feats, targets, mem):
    ids = targets[:, 0]
    feats2 = feats.reshape(B, ROW)
    f_flat, t_flat = _sc_call(feats2, ids)
    return f_flat.reshape(B, S + K, D), t_flat.reshape(B, S + K)
